# baseline (speedup 1.0000x reference)
import functools

import jax
import jax.numpy as jnp
from jax import lax
from jax.experimental import pallas as pl
from jax.experimental.pallas import tpu as pltpu
from jax.experimental.pallas import tpu_sc as plsc

N = 20000
M = 5000
D = 128
MP = 5120
BN = 2000
BM = 512

NC = 2
NS = 16
NW = NC * NS
BPW = MP // NW
HALF = BPW // 2


def _argmin_body(v_ref, s_ref, idx_ref, minv_ref, mini_ref):
    n = pl.program_id(1)
    a = v_ref[...]
    st = s_ref[...]
    a2 = jnp.sum(a * a, axis=1, keepdims=True)
    b2 = jnp.sum(st * st, axis=0, keepdims=True)
    ab = jnp.dot(a, st)
    d2 = a2 + b2 - 2.0 * ab
    dist = jnp.sqrt(jnp.maximum(d2, 0.0))

    bmin = jnp.min(dist, axis=0, keepdims=True)
    rows = lax.broadcasted_iota(jnp.int32, (BN, BM), 0) + n * BN
    bidx = jnp.min(
        jnp.where(dist == bmin, rows, jnp.int32(2**30)),
        axis=0, keepdims=True)

    @pl.when(n == 0)
    def _():
        minv_ref[...] = bmin
        mini_ref[...] = bidx

    @pl.when(n > 0)
    def _():
        better = bmin < minv_ref[...]
        minv_ref[...] = jnp.where(better, bmin, minv_ref[...])
        mini_ref[...] = jnp.where(better, bidx, mini_ref[...])

    @pl.when(n == pl.num_programs(1) - 1)
    def _():
        idx_ref[...] = mini_ref[...]


def _nearest_idx(vertices, s_t):
    return pl.pallas_call(
        _argmin_body,
        grid=(MP // BM, N // BN),
        in_specs=[
            pl.BlockSpec((BN, 3), lambda m, n: (n, 0)),
            pl.BlockSpec((3, BM), lambda m, n: (0, m)),
        ],
        out_specs=pl.BlockSpec((1, BM), lambda m, n: (0, m)),
        out_shape=jax.ShapeDtypeStruct((1, MP), jnp.int32),
        scratch_shapes=[
            pltpu.VMEM((1, BM), jnp.float32),
            pltpu.VMEM((1, BM), jnp.int32),
        ],
    )(vertices, s_t)


@functools.lru_cache(maxsize=None)
def _make_sc_gather():
    @functools.partial(
        pl.kernel,
        mesh=plsc.VectorSubcoreMesh(core_axis_name="c", subcore_axis_name="s"),
        out_type=jax.ShapeDtypeStruct((MP, D), jnp.float32),
        scratch_types=[
            pltpu.VMEM((2, HALF), jnp.int32),
            pltpu.VMEM((BPW, D), jnp.float32),
            pltpu.SemaphoreType.DMA,
        ],
    )
    def _sc_gather(x_hbm, idx_hbm, out_hbm, idx_v, rows_v, sem):
        wid = lax.axis_index("s") * NC + lax.axis_index("c")
        for j in range(2):
            pltpu.sync_copy(idx_hbm.at[2 * wid + j], idx_v.at[j])
            pltpu.async_copy(
                x_hbm.at[idx_v.at[j]], rows_v.at[pl.ds(j * HALF, HALF)], sem
            ).wait()
        pltpu.sync_copy(rows_v, out_hbm.at[pl.ds(wid * BPW, BPW)])

    return _sc_gather


def kernel(vertices, sub_vertices, X):
    s_t = jnp.zeros((3, MP), jnp.float32).at[:, :M].set(sub_vertices.T)
    idx = _nearest_idx(vertices, s_t)
    idx2 = idx.reshape(NW * 2, HALF)
    rows = _make_sc_gather()(X, idx2)
    return rows[:M]


# speedup vs baseline: 1.3703x; 1.3703x over previous
import functools

import jax
import jax.numpy as jnp
from jax import lax
from jax.experimental import pallas as pl
from jax.experimental.pallas import tpu as pltpu
from jax.experimental.pallas import tpu_sc as plsc

N = 20000
M = 5000
D = 128
MP = 5120
BN = 2000
BM = 512

NC = 2
NS = 16
NW = NC * NS
BPW = MP // NW
HALF = BPW // 2


def _argmin_body(v_ref, s_ref, idx_ref, minv_ref, mini_ref):
    n = pl.program_id(1)
    a = v_ref[...]
    st = s_ref[...]
    a2 = 0.25 * jnp.sum(a * a, axis=1, keepdims=True)
    b2 = jnp.sum(st * st, axis=0, keepdims=True)
    ab2 = jnp.dot(a, st)
    d2 = (a2 + b2) - ab2

    bmin = jnp.min(d2, axis=0, keepdims=True)
    bminc = jnp.maximum(bmin, 0.0)
    s = jnp.sqrt(bminc)
    c = s * s
    cb = lax.bitcast_convert_type(c, jnp.int32)
    T = bminc
    for k in range(-4, 5):
        cand = lax.bitcast_convert_type(cb + k, jnp.float32)
        ok = jnp.sqrt(cand) <= s
        T = jnp.where(ok, jnp.maximum(T, cand), T)
    T = jnp.where(s == 0.0, 0.0, T)

    rows = lax.broadcasted_iota(jnp.int32, (BN, BM), 0)
    bidx = jnp.min(
        jnp.where(d2 <= T, rows, jnp.int32(2**30)),
        axis=0, keepdims=True) + n * BN

    @pl.when(n == 0)
    def _():
        minv_ref[...] = s
        mini_ref[...] = bidx

    @pl.when(n > 0)
    def _():
        better = s < minv_ref[...]
        minv_ref[...] = jnp.where(better, s, minv_ref[...])
        mini_ref[...] = jnp.where(better, bidx, mini_ref[...])

    @pl.when(n == pl.num_programs(1) - 1)
    def _():
        idx_ref[...] = mini_ref[...]


def _nearest_idx(vertices, s_t):
    return pl.pallas_call(
        _argmin_body,
        grid=(MP // BM, N // BN),
        in_specs=[
            pl.BlockSpec((BN, 3), lambda m, n: (n, 0)),
            pl.BlockSpec((3, BM), lambda m, n: (0, m)),
        ],
        out_specs=pl.BlockSpec((1, BM), lambda m, n: (0, m)),
        out_shape=jax.ShapeDtypeStruct((1, MP), jnp.int32),
        scratch_shapes=[
            pltpu.VMEM((1, BM), jnp.float32),
            pltpu.VMEM((1, BM), jnp.int32),
        ],
    )(vertices, s_t)


@functools.lru_cache(maxsize=None)
def _make_sc_gather():
    @functools.partial(
        pl.kernel,
        mesh=plsc.VectorSubcoreMesh(core_axis_name="c", subcore_axis_name="s"),
        out_type=jax.ShapeDtypeStruct((MP, D), jnp.float32),
        scratch_types=[
            pltpu.VMEM((2, HALF), jnp.int32),
            pltpu.VMEM((BPW, D), jnp.float32),
            pltpu.SemaphoreType.DMA,
        ],
    )
    def _sc_gather(x_hbm, idx_hbm, out_hbm, idx_v, rows_v, sem):
        wid = lax.axis_index("s") * NC + lax.axis_index("c")
        for j in range(2):
            pltpu.sync_copy(idx_hbm.at[2 * wid + j], idx_v.at[j])
            pltpu.async_copy(
                x_hbm.at[idx_v.at[j]], rows_v.at[pl.ds(j * HALF, HALF)], sem
            ).wait()
        pltpu.sync_copy(rows_v, out_hbm.at[pl.ds(wid * BPW, BPW)])

    return _sc_gather


def kernel(vertices, sub_vertices, X):
    s_t = jnp.zeros((3, MP), jnp.float32).at[:, :M].set(sub_vertices.T)
    idx = _nearest_idx(2.0 * vertices, s_t)
    idx2 = idx.reshape(NW * 2, HALF)
    rows = _make_sc_gather()(X, idx2)
    return rows[:M]


# speedup vs baseline: 1.5105x; 1.1023x over previous
import functools

import jax
import jax.numpy as jnp
from jax import lax
from jax.experimental import pallas as pl
from jax.experimental.pallas import tpu as pltpu
from jax.experimental.pallas import tpu_sc as plsc

N = 20000
NP = 20480
M = 5000
D = 128
MP = 5120
BN = 2048
BM = 1024

NC = 2
NS = 16
NW = NC * NS
BPW = MP // NW
HALF = BPW // 2


def _argmin_body(v_ref, vt_ref, s_ref, idx_ref, a2_ref, minv_ref, mini_ref):
    n = pl.program_id(0)
    m = pl.program_id(1)

    @pl.when(m == 0)
    def _():
        t = vt_ref[...]
        aa = t[0:1] * t[0:1] + t[1:2] * t[1:2] + t[2:3] * t[2:3]
        a2_ref[...] = jnp.transpose(0.25 * aa)

    a2 = a2_ref[...]
    st = s_ref[...]
    b2 = st[0:1] * st[0:1] + st[1:2] * st[1:2] + st[2:3] * st[2:3]
    ab2 = jnp.dot(v_ref[...], st)
    d2 = (a2 + b2) - ab2

    bmin = jnp.min(d2, axis=0, keepdims=True)
    bminc = jnp.maximum(bmin, 0.0)
    s = jnp.sqrt(bminc)
    c = s * s
    cb = lax.bitcast_convert_type(c, jnp.int32)
    T = bminc
    for k in range(-4, 5):
        cand = lax.bitcast_convert_type(cb + k, jnp.float32)
        ok = jnp.sqrt(cand) <= s
        T = jnp.where(ok, jnp.maximum(T, cand), T)
    T = jnp.where(s == 0.0, 0.0, T)

    rows = lax.broadcasted_iota(jnp.int32, (BN, BM), 0)
    bidx = jnp.min(
        jnp.where(d2 <= T, rows, jnp.int32(2**30)),
        axis=0, keepdims=True) + n * BN

    sl = pl.ds(m * BM, BM)

    @pl.when(n == 0)
    def _():
        minv_ref[:, sl] = s
        mini_ref[:, sl] = bidx

    @pl.when(n > 0)
    def _():
        rv = minv_ref[:, sl]
        ri = mini_ref[:, sl]
        better = s < rv
        minv_ref[:, sl] = jnp.where(better, s, rv)
        mini_ref[:, sl] = jnp.where(better, bidx, ri)

    idx_ref[...] = mini_ref[:, sl]


def _nearest_idx(vertices, vertices_t, s_t):
    return pl.pallas_call(
        _argmin_body,
        grid=(NP // BN, MP // BM),
        in_specs=[
            pl.BlockSpec((BN, 3), lambda n, m: (n, 0)),
            pl.BlockSpec((3, BN), lambda n, m: (0, n)),
            pl.BlockSpec((3, BM), lambda n, m: (0, m)),
        ],
        out_specs=pl.BlockSpec((1, BM), lambda n, m: (0, m)),
        out_shape=jax.ShapeDtypeStruct((1, MP), jnp.int32),
        scratch_shapes=[
            pltpu.VMEM((BN, 1), jnp.float32),
            pltpu.VMEM((1, MP), jnp.float32),
            pltpu.VMEM((1, MP), jnp.int32),
        ],
    )(vertices, vertices_t, s_t)


@functools.lru_cache(maxsize=None)
def _make_sc_gather():
    @functools.partial(
        pl.kernel,
        mesh=plsc.VectorSubcoreMesh(core_axis_name="c", subcore_axis_name="s"),
        out_type=jax.ShapeDtypeStruct((MP, D), jnp.float32),
        scratch_types=[
            pltpu.VMEM((2, HALF), jnp.int32),
            pltpu.VMEM((BPW, D), jnp.float32),
            pltpu.SemaphoreType.DMA,
        ],
    )
    def _sc_gather(x_hbm, idx_hbm, out_hbm, idx_v, rows_v, sem):
        wid = lax.axis_index("s") * NC + lax.axis_index("c")
        for j in range(2):
            pltpu.sync_copy(idx_hbm.at[2 * wid + j], idx_v.at[j])
            pltpu.async_copy(
                x_hbm.at[idx_v.at[j]], rows_v.at[pl.ds(j * HALF, HALF)], sem
            ).wait()
        pltpu.sync_copy(rows_v, out_hbm.at[pl.ds(wid * BPW, BPW)])

    return _sc_gather


def kernel(vertices, sub_vertices, X):
    s_t = jnp.zeros((3, MP), jnp.float32).at[:, :M].set(sub_vertices.T)
    v2 = jnp.full((NP, 3), 1.0e18, jnp.float32).at[:N].set(2.0 * vertices)
    idx = _nearest_idx(v2, v2.T, s_t)
    idx2 = idx.reshape(NW * 2, HALF)
    rows = _make_sc_gather()(X, idx2)
    return rows[:M]
